# baseline (speedup 1.0000x reference)
import jax
import jax.numpy as jnp
from jax import lax
from jax.experimental import pallas as pl
from jax.experimental.pallas import tpu as pltpu
from jax.experimental.pallas import tpu_sc as plsc

_N = 10000
_E = 320000
_D = 128
_NC = 2
_NS = 16
_CHUNK = 128
_NCHUNKS = _E // _CHUNK
_HALF = _N // _NC
_SPAD = 5120
_RPT = _SPAD // _NS
_GARB = _SPAD - 1


def _make_sc_pass():
    mesh = plsc.VectorSubcoreMesh(core_axis_name="c", subcore_axis_name="s", num_cores=_NC, num_subcores=_NS)
    out_type = jax.ShapeDtypeStruct((_NC * _SPAD, _D), jnp.float32)
    scratch = [
        pltpu.VMEM((_CHUNK,), jnp.int32),
        pltpu.VMEM((1, _CHUNK), jnp.int32),
        pltpu.VMEM((1, _CHUNK), jnp.int32),
        pltpu.VMEM((_CHUNK, _D), jnp.float32),
        pltpu.VMEM((_RPT, _D), jnp.float32),
        pltpu.VMEM_SHARED((_SPAD, _D), jnp.float32),
        pltpu.SemaphoreType.DMA,
    ]

    def body(feats, src, dst, agg_out, src_idx, dst_idx, ldst_idx,
             rows, stage, agg_sh, sem):
        c = lax.axis_index("c")
        s = lax.axis_index("s")
        row0 = s * _RPT
        lo = c * _HALF

        zero16 = jnp.zeros((16,), jnp.float32)

        def zrow(i, carry):
            for j in range(_D // 16):
                stage[i, pl.ds(j * 16, 16)] = zero16
            return carry

        lax.fori_loop(0, _RPT, zrow, 0)
        pltpu.sync_copy(stage, agg_sh.at[pl.ds(row0, _RPT)])
        plsc.subcore_barrier()

        nloc = (_NCHUNKS - s + _NS - 1) // _NS

        def step(i, carry):
            base = (s + i * _NS) * _CHUNK
            pltpu.sync_copy(src.at[pl.ds(base, _CHUNK)], src_idx)
            pltpu.sync_copy(dst.at[pl.ds(base, _CHUNK)], dst_idx.at[0])
            gather = pltpu.async_copy(feats.at[src_idx], rows, sem)
            for j in range(_CHUNK // 16):
                d = dst_idx[0, pl.ds(j * 16, 16)]
                keep = (d >= lo) & (d < lo + _HALF)
                ldst_idx[0, pl.ds(j * 16, 16)] = jnp.where(
                    keep, d - lo, jnp.full((16,), _GARB, jnp.int32))
            gather.wait()
            pltpu.sync_copy(rows, agg_sh.at[ldst_idx.at[0]], add=True)
            return carry

        lax.fori_loop(0, nloc, step, 0)
        plsc.subcore_barrier()

        off = c * _SPAD + row0
        pltpu.sync_copy(agg_sh.at[pl.ds(row0, _RPT)], stage)
        pltpu.sync_copy(stage, agg_out.at[pl.ds(off, _RPT)])

    return pl.kernel(body, out_type=out_type, mesh=mesh,
                     scratch_types=scratch)


def _make_cnt_pass():
    mesh = plsc.VectorSubcoreMesh(core_axis_name="c", subcore_axis_name="s", num_cores=_NC, num_subcores=_NS)
    out_type = jax.ShapeDtypeStruct((_NC * _SPAD, 16), jnp.float32)
    scratch = [
        pltpu.VMEM((1, _CHUNK), jnp.int32),
        pltpu.VMEM((1, _CHUNK), jnp.int32),
        pltpu.VMEM((_CHUNK, 16), jnp.float32),
        pltpu.VMEM((_RPT, 16), jnp.float32),
        pltpu.VMEM_SHARED((_SPAD, 16), jnp.float32),
        pltpu.SemaphoreType.DMA,
    ]

    def body(dst, cnt_out, dst_idx, ldst_idx, ones, cstage, cnt_sh, sem):
        c = lax.axis_index("c")
        s = lax.axis_index("s")
        row0 = s * _RPT
        lo = c * _HALF

        zero16 = jnp.zeros((16,), jnp.float32)
        one16 = jnp.ones((16,), jnp.float32)

        def zrow(i, carry):
            cstage[i, :] = zero16
            return carry

        lax.fori_loop(0, _RPT, zrow, 0)

        def orow(i, carry):
            ones[i, :] = one16
            return carry

        lax.fori_loop(0, _CHUNK, orow, 0)

        pltpu.sync_copy(cstage, cnt_sh.at[pl.ds(row0, _RPT)])
        plsc.subcore_barrier()

        nloc = (_NCHUNKS - s + _NS - 1) // _NS

        def step(i, carry):
            base = (s + i * _NS) * _CHUNK
            pltpu.sync_copy(dst.at[pl.ds(base, _CHUNK)], dst_idx.at[0])
            for j in range(_CHUNK // 16):
                d = dst_idx[0, pl.ds(j * 16, 16)]
                keep = (d >= lo) & (d < lo + _HALF)
                ldst_idx[0, pl.ds(j * 16, 16)] = jnp.where(
                    keep, d - lo, jnp.full((16,), _GARB, jnp.int32))
            pltpu.sync_copy(ones, cnt_sh.at[ldst_idx.at[0]], add=True)
            return carry

        lax.fori_loop(0, nloc, step, 0)
        plsc.subcore_barrier()

        off = c * _SPAD + row0
        pltpu.sync_copy(cnt_sh.at[pl.ds(row0, _RPT)], cstage)
        pltpu.sync_copy(cstage, cnt_out.at[pl.ds(off, _RPT)])

    return pl.kernel(body, out_type=out_type, mesh=mesh,
                     scratch_types=scratch)


_sc_pass = _make_sc_pass()
_cnt_pass = _make_cnt_pass()

_R = 1000


def _layer_body(agg_ref, cnt_ref, x_ref, wl_ref, wr_ref, b_ref, o_ref):
    cnt = cnt_ref[:, 0:1]
    mean = agg_ref[...] / jnp.maximum(cnt, 1.0)
    acc = jnp.dot(mean, wl_ref[...], preferred_element_type=jnp.float32,
                  precision=lax.Precision.HIGHEST)
    acc = acc + jnp.dot(x_ref[...], wr_ref[...],
                        preferred_element_type=jnp.float32,
                        precision=lax.Precision.HIGHEST)
    o_ref[...] = jnp.maximum(acc + b_ref[...], 0.0)


def _final_body(agg_ref, cnt_ref, h_ref, wl_ref, wr_ref, bl_ref,
                w3_ref, b3_ref, w4_ref, b4_ref, o_ref):
    cnt = cnt_ref[:, 0:1]
    mean = agg_ref[...] / jnp.maximum(cnt, 1.0)
    h = jnp.dot(mean, wl_ref[...], preferred_element_type=jnp.float32,
                precision=lax.Precision.HIGHEST)
    h = h + jnp.dot(h_ref[...], wr_ref[...],
                    preferred_element_type=jnp.float32,
                    precision=lax.Precision.HIGHEST)
    h = jnp.maximum(h + bl_ref[...], 0.0)
    h = jnp.maximum(
        jnp.dot(h, w3_ref[...], preferred_element_type=jnp.float32,
                precision=lax.Precision.HIGHEST) + b3_ref[...], 0.0)
    z = jnp.dot(h, w4_ref[...], preferred_element_type=jnp.float32,
                precision=lax.Precision.HIGHEST) + b4_ref[...]
    z = z - jnp.max(z, axis=-1, keepdims=True)
    e = jnp.exp(z)
    o_ref[...] = e / jnp.sum(e, axis=-1, keepdims=True)


def _full_spec():
    return pl.BlockSpec((_D, _D), lambda i: (0, 0))


def _bias_spec():
    return pl.BlockSpec((1, _D), lambda i: (0, 0))


def _tc_layer(agg, cnt16, feats, Wl, Wr, bl):
    return pl.pallas_call(
        _layer_body,
        grid=(_N // _R,),
        in_specs=[
            pl.BlockSpec((_R, _D), lambda i: (i, 0)),
            pl.BlockSpec((_R, 16), lambda i: (i, 0)),
            pl.BlockSpec((_R, _D), lambda i: (i, 0)),
            _full_spec(), _full_spec(), _bias_spec(),
        ],
        out_specs=pl.BlockSpec((_R, _D), lambda i: (i, 0)),
        out_shape=jax.ShapeDtypeStruct((_N, _D), jnp.float32),
    )(agg, cnt16, feats, Wl, Wr, bl)


def _tc_final(agg, cnt16, h1, Wl, Wr, bl, W3, b3, W4, b4):
    return pl.pallas_call(
        _final_body,
        grid=(_N // _R,),
        in_specs=[
            pl.BlockSpec((_R, _D), lambda i: (i, 0)),
            pl.BlockSpec((_R, 16), lambda i: (i, 0)),
            pl.BlockSpec((_R, _D), lambda i: (i, 0)),
            _full_spec(), _full_spec(), _bias_spec(),
            _full_spec(), _bias_spec(),
            _full_spec(), _bias_spec(),
        ],
        out_specs=pl.BlockSpec((_R, _D), lambda i: (i, 0)),
        out_shape=jax.ShapeDtypeStruct((_N, _D), jnp.float32),
    )(agg, cnt16, h1, Wl, Wr, bl, W3, b3, W4, b4)


def _unpad(a):
    return jnp.concatenate([a[:_HALF], a[_SPAD:_SPAD + _HALF]], axis=0)


def kernel(x, edge_index, Wl1, bl1, Wr1, Wl2, bl2, Wr2, W3, b3, W4, b4):
    src = edge_index[0]
    dst = edge_index[1]

    agg1 = _unpad(_sc_pass(x, src, dst))
    cnt = _unpad(_cnt_pass(dst))
    h1 = _tc_layer(agg1, cnt, x, Wl1, Wr1, bl1.reshape(1, _D))

    agg2 = _unpad(_sc_pass(h1, src, dst))
    return _tc_final(agg2, cnt, h1, Wl2, Wr2, bl2.reshape(1, _D),
                     W3, b3.reshape(1, _D), W4, b4.reshape(1, _D))


# speedup vs baseline: 1.2376x; 1.2376x over previous
import jax
import jax.numpy as jnp
from jax import lax
from jax.experimental import pallas as pl
from jax.experimental.pallas import tpu as pltpu
from jax.experimental.pallas import tpu_sc as plsc

_N = 10000
_E = 320000
_D = 128
_NC = 2
_NS = 16
_CHUNK = 128
_NCHUNKS = _E // _CHUNK
_HALF = _N // _NC
_SPAD = 5120
_RPT = _SPAD // _NS
_GARB = _SPAD - 1
_NW = _NC * _NS
_CAPB = 160
_CAP = _CAPB * _CHUNK


def _mesh():
    return plsc.VectorSubcoreMesh(core_axis_name="c", subcore_axis_name="s",
                                  num_cores=_NC, num_subcores=_NS)


def _make_route():
    out_type = (
        jax.ShapeDtypeStruct((_NW * _CAPB, _CHUNK), jnp.int32),
        jax.ShapeDtypeStruct((_NW * 16,), jnp.int32),
    )
    scratch = [
        pltpu.VMEM((1, _CHUNK), jnp.int32),
        pltpu.VMEM((1, _CHUNK), jnp.int32),
        pltpu.VMEM((_CAPB, _CHUNK), jnp.int32),
        pltpu.VMEM((16,), jnp.int32),
    ]

    def body(src, dst, fpk_out, nb_out, src_idx, dst_idx, fpk, nbv):
        c = lax.axis_index("c")
        s = lax.axis_index("s")
        wid = c * _NS + s
        lo = c * _HALF

        iota16 = lax.iota(jnp.int32, 16)
        nloc = (_NCHUNKS - s + _NS - 1) // _NS

        def scan(i, fill):
            base = (s + i * _NS) * _CHUNK
            pltpu.sync_copy(src.at[pl.ds(base, _CHUNK)], src_idx.at[0])
            pltpu.sync_copy(dst.at[pl.ds(base, _CHUNK)], dst_idx.at[0])
            for j in range(_CHUNK // 16):
                sv = src_idx[0, pl.ds(j * 16, 16)]
                dv = dst_idx[0, pl.ds(j * 16, 16)]
                keep = (dv >= lo) & (dv < lo + _HALF)
                kint = keep.astype(jnp.int32)
                csum = plsc.cumsum(kint)
                pos = fill + csum - kint
                packed = sv | lax.shift_left(dv - lo, 14)
                plsc.store_scatter(
                    fpk, [lax.shift_right_logical(pos, 7), pos & 127],
                    packed, mask=keep)
                fill = fill + jnp.max(csum)
            return fill

        fill = lax.fori_loop(0, nloc, scan, jnp.int32(0))

        padval16 = jnp.full((16,), _GARB << 14, jnp.int32)
        target = ((fill + _CHUNK - 1) // _CHUNK) * _CHUNK
        r1 = (16 - lax.rem(fill, 16)) & 15
        head = iota16 < r1
        ppos = fill + iota16
        plsc.store_scatter(
            fpk, [lax.shift_right_logical(ppos, 7), ppos & 127],
            padval16, mask=head)
        fill = fill + r1
        full16 = iota16 < 16

        def pad16(i, f):
            qpos = f + iota16
            plsc.store_scatter(
                fpk, [lax.shift_right_logical(qpos, 7), qpos & 127],
                padval16, mask=full16)
            return f + 16

        lax.fori_loop(0, (target - fill) // 16, pad16, fill)

        nbv[...] = jnp.full((16,), target // _CHUNK, jnp.int32)
        pltpu.sync_copy(nbv, nb_out.at[pl.ds(wid * 16, 16)])
        pltpu.sync_copy(fpk, fpk_out.at[pl.ds(wid * _CAPB, _CAPB)])

    return pl.kernel(body, out_type=out_type, mesh=_mesh(),
                     scratch_types=scratch,
                     compiler_params=pltpu.CompilerParams(
                         needs_layout_passes=False))


def _make_agg(with_cnt):
    out_type = [jax.ShapeDtypeStruct((_NC * _SPAD, _D), jnp.float32)]
    scratch = [
        pltpu.VMEM((_CAPB, _CHUNK), jnp.int32),
        pltpu.VMEM((_CHUNK,), jnp.int32),
        pltpu.VMEM((1, _CHUNK), jnp.int32),
        pltpu.VMEM((_CHUNK, _D), jnp.float32),
        pltpu.VMEM((_RPT, _D), jnp.float32),
        pltpu.VMEM((16,), jnp.int32),
        pltpu.VMEM_SHARED((_SPAD, _D), jnp.float32),
        pltpu.SemaphoreType.DMA,
    ]
    if with_cnt:
        out_type.append(jax.ShapeDtypeStruct((_NC * _SPAD, 16), jnp.float32))
        scratch += [
            pltpu.VMEM((_CHUNK, 16), jnp.float32),
            pltpu.VMEM((_RPT, 16), jnp.float32),
            pltpu.VMEM_SHARED((_SPAD, 16), jnp.float32),
        ]

    def body(feats, fpk_all, nb_all, *refs):
        if with_cnt:
            (agg_out, cnt_out, fpk, bat_src, bat_idx, rows, stage, nbv,
             agg_sh, sem, ones, cstage, cnt_sh) = refs
        else:
            (agg_out, fpk, bat_src, bat_idx, rows, stage, nbv,
             agg_sh, sem) = refs
        c = lax.axis_index("c")
        s = lax.axis_index("s")
        wid = c * _NS + s
        row0 = s * _RPT

        zero16 = jnp.zeros((16,), jnp.float32)

        def zrow(i, carry):
            for j in range(_D // 16):
                stage[i, pl.ds(j * 16, 16)] = zero16
            return carry

        lax.fori_loop(0, _RPT, zrow, 0)
        pltpu.sync_copy(stage, agg_sh.at[pl.ds(row0, _RPT)])
        if with_cnt:
            one16 = jnp.ones((16,), jnp.float32)

            def orow(i, carry):
                ones[i, :] = one16
                return carry

            lax.fori_loop(0, _CHUNK, orow, 0)

            def czrow(i, carry):
                cstage[i, :] = zero16
                return carry

            lax.fori_loop(0, _RPT, czrow, 0)
            pltpu.sync_copy(cstage, cnt_sh.at[pl.ds(row0, _RPT)])
        plsc.subcore_barrier()

        pltpu.sync_copy(nb_all.at[pl.ds(wid * 16, 16)], nbv)
        nb = nbv[pl.ds(0, 16)][0]
        iota16 = lax.iota(jnp.int32, 16)
        for g in range(_CHUNK // 16):
            bat_src[pl.ds(g * 16, 16)] = wid * _CAPB + g * 16 + iota16
        pltpu.async_copy(
            fpk_all.at[bat_src], fpk.at[pl.ds(0, _CHUNK)], sem).wait()
        for g in range((_CAPB - _CHUNK) // 16):
            bat_src[pl.ds(g * 16, 16)] = (wid * _CAPB + _CHUNK
                                          + g * 16 + iota16)
        pltpu.async_copy(
            fpk_all.at[bat_src.at[pl.ds(0, _CAPB - _CHUNK)]],
            fpk.at[pl.ds(_CHUNK, _CAPB - _CHUNK)], sem).wait()
        mask14 = jnp.full((16,), 0x3FFF, jnp.int32)

        def flush(k, carry):
            for j in range(_CHUNK // 16):
                w = fpk[k, pl.ds(j * 16, 16)]
                bat_src[pl.ds(j * 16, 16)] = w & mask14
                bat_idx[0, pl.ds(j * 16, 16)] = lax.shift_right_logical(w, 14)
            gather = pltpu.async_copy(feats.at[bat_src], rows, sem)
            if with_cnt:
                pltpu.sync_copy(ones, cnt_sh.at[bat_idx.at[0]], add=True)
            gather.wait()
            pltpu.sync_copy(rows, agg_sh.at[bat_idx.at[0]], add=True)
            return carry

        lax.fori_loop(0, nb, flush, 0)
        plsc.subcore_barrier()

        off = c * _SPAD + row0
        pltpu.sync_copy(agg_sh.at[pl.ds(row0, _RPT)], stage)
        pltpu.sync_copy(stage, agg_out.at[pl.ds(off, _RPT)])
        if with_cnt:
            pltpu.sync_copy(cnt_sh.at[pl.ds(row0, _RPT)], cstage)
            pltpu.sync_copy(cstage, cnt_out.at[pl.ds(off, _RPT)])

    return pl.kernel(body, out_type=tuple(out_type), mesh=_mesh(),
                     scratch_types=scratch)


_route = _make_route()
_aggk = _make_agg(False)

_R = 1000


def _layer_body(agg_ref, cnt_ref, x_ref, wl_ref, wr_ref, b_ref, o_ref):
    cnt = cnt_ref[:, 0:1]
    mean = agg_ref[...] / jnp.maximum(cnt, 1.0)
    acc = jnp.dot(mean, wl_ref[...], preferred_element_type=jnp.float32,
                  precision=lax.Precision.HIGHEST)
    acc = acc + jnp.dot(x_ref[...], wr_ref[...],
                        preferred_element_type=jnp.float32,
                        precision=lax.Precision.HIGHEST)
    o_ref[...] = jnp.maximum(acc + b_ref[...], 0.0)


def _final_body(agg_ref, cnt_ref, h_ref, wl_ref, wr_ref, bl_ref,
                w3_ref, b3_ref, w4_ref, b4_ref, o_ref):
    cnt = cnt_ref[:, 0:1]
    mean = agg_ref[...] / jnp.maximum(cnt, 1.0)
    h = jnp.dot(mean, wl_ref[...], preferred_element_type=jnp.float32,
                precision=lax.Precision.HIGHEST)
    h = h + jnp.dot(h_ref[...], wr_ref[...],
                    preferred_element_type=jnp.float32,
                    precision=lax.Precision.HIGHEST)
    h = jnp.maximum(h + bl_ref[...], 0.0)
    h = jnp.maximum(
        jnp.dot(h, w3_ref[...], preferred_element_type=jnp.float32,
                precision=lax.Precision.HIGHEST) + b3_ref[...], 0.0)
    z = jnp.dot(h, w4_ref[...], preferred_element_type=jnp.float32,
                precision=lax.Precision.HIGHEST) + b4_ref[...]
    z = z - jnp.max(z, axis=-1, keepdims=True)
    e = jnp.exp(z)
    o_ref[...] = e / jnp.sum(e, axis=-1, keepdims=True)


def _full_spec():
    return pl.BlockSpec((_D, _D), lambda i: (0, 0))


def _bias_spec():
    return pl.BlockSpec((1, _D), lambda i: (0, 0))


def _tc_layer(agg, cnt16, feats, Wl, Wr, bl):
    return pl.pallas_call(
        _layer_body,
        grid=(_N // _R,),
        in_specs=[
            pl.BlockSpec((_R, _D), lambda i: (i, 0)),
            pl.BlockSpec((_R, _D), lambda i: (i, 0)),
            pl.BlockSpec((_R, _D), lambda i: (i, 0)),
            _full_spec(), _full_spec(), _bias_spec(),
        ],
        out_specs=pl.BlockSpec((_R, _D), lambda i: (i, 0)),
        out_shape=jax.ShapeDtypeStruct((_N, _D), jnp.float32),
    )(agg, cnt16, feats, Wl, Wr, bl)


def _tc_final(agg, cnt16, h1, Wl, Wr, bl, W3, b3, W4, b4):
    return pl.pallas_call(
        _final_body,
        grid=(_N // _R,),
        in_specs=[
            pl.BlockSpec((_R, _D), lambda i: (i, 0)),
            pl.BlockSpec((_R, _D), lambda i: (i, 0)),
            pl.BlockSpec((_R, _D), lambda i: (i, 0)),
            _full_spec(), _full_spec(), _bias_spec(),
            _full_spec(), _bias_spec(),
            _full_spec(), _bias_spec(),
        ],
        out_specs=pl.BlockSpec((_R, _D), lambda i: (i, 0)),
        out_shape=jax.ShapeDtypeStruct((_N, _D), jnp.float32),
    )(agg, cnt16, h1, Wl, Wr, bl, W3, b3, W4, b4)


def _unpad(a):
    return jnp.concatenate([a[:_HALF], a[_SPAD:_SPAD + _HALF]], axis=0)


def kernel(x, edge_index, Wl1, bl1, Wr1, Wl2, bl2, Wr2, W3, b3, W4, b4):
    src = edge_index[0]
    dst = edge_index[1]

    fpk_all, nb_all = _route(src, dst)

    ones_feats = jnp.ones((_N, _D), jnp.float32)
    cnt = _unpad(_aggk(ones_feats, fpk_all, nb_all)[0])
    agg1 = _unpad(_aggk(x, fpk_all, nb_all)[0])
    h1 = _tc_layer(agg1, cnt, x, Wl1, Wr1, bl1.reshape(1, _D))

    agg2 = _unpad(_aggk(h1, fpk_all, nb_all)[0])
    return _tc_final(agg2, cnt, h1, Wl2, Wr2, bl2.reshape(1, _D),
                     W3, b3.reshape(1, _D), W4, b4.reshape(1, _D))


# speedup vs baseline: 1.2376x; 1.0000x over previous
import jax
import jax.numpy as jnp
from jax import lax
from jax.experimental import pallas as pl
from jax.experimental.pallas import tpu as pltpu
from jax.experimental.pallas import tpu_sc as plsc

_N = 10000
_E = 320000
_D = 128
_NC = 2
_NS = 16
_CHUNK = 128
_NCHUNKS = _E // _CHUNK
_HALF = _N // _NC
_SPAD = 5120
_RPT = _SPAD // _NS
_GARB = _SPAD - 1
_NW = _NC * _NS
_CAPB = 160
_CAP = _CAPB * _CHUNK


def _mesh():
    return plsc.VectorSubcoreMesh(core_axis_name="c", subcore_axis_name="s",
                                  num_cores=_NC, num_subcores=_NS)


def _make_route():
    out_type = (
        jax.ShapeDtypeStruct((_NW * _CAPB, _CHUNK), jnp.int32),
        jax.ShapeDtypeStruct((_NW * 16,), jnp.int32),
    )
    scratch = [
        pltpu.VMEM((1, _CHUNK), jnp.int32),
        pltpu.VMEM((1, _CHUNK), jnp.int32),
        pltpu.VMEM((_CAPB, _CHUNK), jnp.int32),
        pltpu.VMEM((16,), jnp.int32),
    ]

    def body(src, dst, fpk_out, nb_out, src_idx, dst_idx, fpk, nbv):
        c = lax.axis_index("c")
        s = lax.axis_index("s")
        wid = c * _NS + s
        lo = c * _HALF

        iota16 = lax.iota(jnp.int32, 16)
        nloc = (_NCHUNKS - s + _NS - 1) // _NS

        def scan(i, fill):
            base = (s + i * _NS) * _CHUNK
            pltpu.sync_copy(src.at[pl.ds(base, _CHUNK)], src_idx.at[0])
            pltpu.sync_copy(dst.at[pl.ds(base, _CHUNK)], dst_idx.at[0])
            for j in range(_CHUNK // 16):
                sv = src_idx[0, pl.ds(j * 16, 16)]
                dv = dst_idx[0, pl.ds(j * 16, 16)]
                keep = (dv >= lo) & (dv < lo + _HALF)
                kint = keep.astype(jnp.int32)
                csum = plsc.cumsum(kint)
                pos = fill + csum - kint
                packed = sv | lax.shift_left(dv - lo, 14)
                plsc.store_scatter(
                    fpk, [lax.shift_right_logical(pos, 7), pos & 127],
                    packed, mask=keep)
                fill = fill + jnp.max(csum)
            return fill

        fill = lax.fori_loop(0, nloc, scan, jnp.int32(0))

        padval16 = jnp.full((16,), _GARB << 14, jnp.int32)
        target = ((fill + _CHUNK - 1) // _CHUNK) * _CHUNK
        r1 = (16 - lax.rem(fill, 16)) & 15
        head = iota16 < r1
        ppos = fill + iota16
        plsc.store_scatter(
            fpk, [lax.shift_right_logical(ppos, 7), ppos & 127],
            padval16, mask=head)
        fill = fill + r1
        full16 = iota16 < 16

        def pad16(i, f):
            qpos = f + iota16
            plsc.store_scatter(
                fpk, [lax.shift_right_logical(qpos, 7), qpos & 127],
                padval16, mask=full16)
            return f + 16

        lax.fori_loop(0, (target - fill) // 16, pad16, fill)

        nbv[...] = jnp.full((16,), target // _CHUNK, jnp.int32)
        pltpu.sync_copy(nbv, nb_out.at[pl.ds(wid * 16, 16)])
        pltpu.sync_copy(fpk, fpk_out.at[pl.ds(wid * _CAPB, _CAPB)])

    return pl.kernel(body, out_type=out_type, mesh=_mesh(),
                     scratch_types=scratch,
                     compiler_params=pltpu.CompilerParams(
                         needs_layout_passes=False))


def _make_agg(with_cnt):
    out_type = [jax.ShapeDtypeStruct((_NC * _SPAD, _D), jnp.float32)]
    scratch = [
        pltpu.VMEM((_CAPB, _CHUNK), jnp.int32),
        pltpu.VMEM((_CHUNK,), jnp.int32),
        pltpu.VMEM((1, _CHUNK), jnp.int32),
        pltpu.VMEM((_CHUNK,), jnp.int32),
        pltpu.VMEM((1, _CHUNK), jnp.int32),
        pltpu.VMEM((_CHUNK, _D), jnp.float32),
        pltpu.VMEM((_CHUNK, _D), jnp.float32),
        pltpu.VMEM((_RPT, _D), jnp.float32),
        pltpu.VMEM((16,), jnp.int32),
        pltpu.VMEM_SHARED((_SPAD, _D), jnp.float32),
        pltpu.SemaphoreType.DMA,
        pltpu.SemaphoreType.DMA,
        pltpu.SemaphoreType.DMA,
    ]
    if with_cnt:
        out_type.append(jax.ShapeDtypeStruct((_NC * _SPAD, 16), jnp.float32))
        scratch += [
            pltpu.VMEM((_CHUNK, 16), jnp.float32),
            pltpu.VMEM((_RPT, 16), jnp.float32),
            pltpu.VMEM_SHARED((_SPAD, 16), jnp.float32),
        ]

    def body(feats, fpk_all, nb_all, *refs):
        if with_cnt:
            (agg_out, cnt_out, fpk, bat_src, bat_idx, rows, stage, nbv,
             agg_sh, sem, ones, cstage, cnt_sh) = refs
        else:
            (agg_out, fpk, bat_src, bat_idx, bat_srcb, bat_idxb, rows, rowsb,
             stage, nbv, agg_sh, sem, semb, sems) = refs
        c = lax.axis_index("c")
        s = lax.axis_index("s")
        wid = c * _NS + s
        row0 = s * _RPT

        zero16 = jnp.zeros((16,), jnp.float32)

        def zrow(i, carry):
            for j in range(_D // 16):
                stage[i, pl.ds(j * 16, 16)] = zero16
            return carry

        lax.fori_loop(0, _RPT, zrow, 0)
        pltpu.sync_copy(stage, agg_sh.at[pl.ds(row0, _RPT)])
        if with_cnt:
            one16 = jnp.ones((16,), jnp.float32)

            def orow(i, carry):
                ones[i, :] = one16
                return carry

            lax.fori_loop(0, _CHUNK, orow, 0)

            def czrow(i, carry):
                cstage[i, :] = zero16
                return carry

            lax.fori_loop(0, _RPT, czrow, 0)
            pltpu.sync_copy(cstage, cnt_sh.at[pl.ds(row0, _RPT)])
        plsc.subcore_barrier()

        pltpu.sync_copy(nb_all.at[pl.ds(wid * 16, 16)], nbv)
        nb = nbv[pl.ds(0, 16)][0]
        iota16 = lax.iota(jnp.int32, 16)
        for g in range(_CHUNK // 16):
            bat_src[pl.ds(g * 16, 16)] = wid * _CAPB + g * 16 + iota16
        pltpu.async_copy(
            fpk_all.at[bat_src], fpk.at[pl.ds(0, _CHUNK)], sem).wait()
        for g in range((_CAPB - _CHUNK) // 16):
            bat_src[pl.ds(g * 16, 16)] = (wid * _CAPB + _CHUNK
                                          + g * 16 + iota16)
        pltpu.async_copy(
            fpk_all.at[bat_src.at[pl.ds(0, _CAPB - _CHUNK)]],
            fpk.at[pl.ds(_CHUNK, _CAPB - _CHUNK)], sem).wait()
        mask14 = jnp.full((16,), 0x3FFF, jnp.int32)

        def flush(k, carry):
            for j in range(_CHUNK // 16):
                w = fpk[k, pl.ds(j * 16, 16)]
                bat_src[pl.ds(j * 16, 16)] = w & mask14
                bat_idx[0, pl.ds(j * 16, 16)] = lax.shift_right_logical(w, 14)
            gather = pltpu.async_copy(feats.at[bat_src], rows, sem)
            if with_cnt:
                pltpu.sync_copy(ones, cnt_sh.at[bat_idx.at[0]], add=True)
            gather.wait()
            pltpu.sync_copy(rows, agg_sh.at[bat_idx.at[0]], add=True)
            return carry

        lax.fori_loop(0, nb, flush, 0)
        plsc.subcore_barrier()

        off = c * _SPAD + row0
        pltpu.sync_copy(agg_sh.at[pl.ds(row0, _RPT)], stage)
        pltpu.sync_copy(stage, agg_out.at[pl.ds(off, _RPT)])
        if with_cnt:
            pltpu.sync_copy(cnt_sh.at[pl.ds(row0, _RPT)], cstage)
            pltpu.sync_copy(cstage, cnt_out.at[pl.ds(off, _RPT)])

    return pl.kernel(body, out_type=tuple(out_type), mesh=_mesh(),
                     scratch_types=scratch)


_route = _make_route()
_aggk = _make_agg(False)

_R = 1000


def _layer_body(agg_ref, cnt_ref, x_ref, wl_ref, wr_ref, b_ref, o_ref):
    cnt = cnt_ref[:, 0:1]
    mean = agg_ref[...] / jnp.maximum(cnt, 1.0)
    acc = jnp.dot(mean, wl_ref[...], preferred_element_type=jnp.float32,
                  precision=lax.Precision.HIGHEST)
    acc = acc + jnp.dot(x_ref[...], wr_ref[...],
                        preferred_element_type=jnp.float32,
                        precision=lax.Precision.HIGHEST)
    o_ref[...] = jnp.maximum(acc + b_ref[...], 0.0)


def _final_body(agg_ref, cnt_ref, h_ref, wl_ref, wr_ref, bl_ref,
                w3_ref, b3_ref, w4_ref, b4_ref, o_ref):
    cnt = cnt_ref[:, 0:1]
    mean = agg_ref[...] / jnp.maximum(cnt, 1.0)
    h = jnp.dot(mean, wl_ref[...], preferred_element_type=jnp.float32,
                precision=lax.Precision.HIGHEST)
    h = h + jnp.dot(h_ref[...], wr_ref[...],
                    preferred_element_type=jnp.float32,
                    precision=lax.Precision.HIGHEST)
    h = jnp.maximum(h + bl_ref[...], 0.0)
    h = jnp.maximum(
        jnp.dot(h, w3_ref[...], preferred_element_type=jnp.float32,
                precision=lax.Precision.HIGHEST) + b3_ref[...], 0.0)
    z = jnp.dot(h, w4_ref[...], preferred_element_type=jnp.float32,
                precision=lax.Precision.HIGHEST) + b4_ref[...]
    z = z - jnp.max(z, axis=-1, keepdims=True)
    e = jnp.exp(z)
    o_ref[...] = e / jnp.sum(e, axis=-1, keepdims=True)


def _full_spec():
    return pl.BlockSpec((_D, _D), lambda i: (0, 0))


def _bias_spec():
    return pl.BlockSpec((1, _D), lambda i: (0, 0))


def _tc_layer(agg, cnt16, feats, Wl, Wr, bl):
    return pl.pallas_call(
        _layer_body,
        grid=(_N // _R,),
        in_specs=[
            pl.BlockSpec((_R, _D), lambda i: (i, 0)),
            pl.BlockSpec((_R, _D), lambda i: (i, 0)),
            pl.BlockSpec((_R, _D), lambda i: (i, 0)),
            _full_spec(), _full_spec(), _bias_spec(),
        ],
        out_specs=pl.BlockSpec((_R, _D), lambda i: (i, 0)),
        out_shape=jax.ShapeDtypeStruct((_N, _D), jnp.float32),
    )(agg, cnt16, feats, Wl, Wr, bl)


def _tc_final(agg, cnt16, h1, Wl, Wr, bl, W3, b3, W4, b4):
    return pl.pallas_call(
        _final_body,
        grid=(_N // _R,),
        in_specs=[
            pl.BlockSpec((_R, _D), lambda i: (i, 0)),
            pl.BlockSpec((_R, _D), lambda i: (i, 0)),
            pl.BlockSpec((_R, _D), lambda i: (i, 0)),
            _full_spec(), _full_spec(), _bias_spec(),
            _full_spec(), _bias_spec(),
            _full_spec(), _bias_spec(),
        ],
        out_specs=pl.BlockSpec((_R, _D), lambda i: (i, 0)),
        out_shape=jax.ShapeDtypeStruct((_N, _D), jnp.float32),
    )(agg, cnt16, h1, Wl, Wr, bl, W3, b3, W4, b4)


def _unpad(a):
    return jnp.concatenate([a[:_HALF], a[_SPAD:_SPAD + _HALF]], axis=0)


def kernel(x, edge_index, Wl1, bl1, Wr1, Wl2, bl2, Wr2, W3, b3, W4, b4):
    src = edge_index[0]
    dst = edge_index[1]

    fpk_all, nb_all = _route(src, dst)

    ones_feats = jnp.ones((_N, _D), jnp.float32)
    cnt = _unpad(_aggk(ones_feats, fpk_all, nb_all)[0])
    agg1 = _unpad(_aggk(x, fpk_all, nb_all)[0])
    h1 = _tc_layer(agg1, cnt, x, Wl1, Wr1, bl1.reshape(1, _D))

    agg2 = _unpad(_aggk(h1, fpk_all, nb_all)[0])
    return _tc_final(agg2, cnt, h1, Wl2, Wr2, bl2.reshape(1, _D),
                     W3, b3.reshape(1, _D), W4, b4.reshape(1, _D))
